# Initial kernel scaffold; baseline (speedup 1.0000x reference)
import jax
import jax.numpy as jnp
from jax.experimental import pallas as pl


def kernel(teacher_feats, student_feats):
    raise NotImplementedError("write your pallas kernel here")


# speedup vs baseline: 2.5004x; 2.5004x over previous
import jax
import jax.numpy as jnp
from jax.experimental import pallas as pl
from jax.experimental.pallas import tpu as pltpu

_STUDENT_IDX = [0, 2, 4, 6]
_SHARED_PAIRS = [(2, 1), (4, 2), (6, 3)]
_TOPK = 4
_N = 256
_BETA = 0.5


def _huber_sum(d):
    ad = jnp.abs(d)
    return jnp.sum(jnp.where(ad < _BETA, 0.5 * d * d / _BETA, ad - 0.5 * _BETA))


def _loss_kernel(rp_ref, sp_ref, t0_ref, s0_ref,
                 t_a_ref, s_a_ref, t_b_ref, s_b_ref, t_c_ref, s_c_ref,
                 out_ref, acc_ref):
    i = pl.program_id(0)

    @pl.when(i == 0)
    def _():
        acc_ref[0] = 0.0

    rt = t0_ref[0, 0]
    rs = s0_ref[0, 0]

    B = rt.shape[0]
    D = rt.shape[1]
    n_d1 = 3 * B * _N
    n_d2 = 3 * B * _N * _TOPK
    w_d1 = 1.0 / (n_d1 * D)
    w_d2 = 12.0 / (n_d2 * D)
    w_d3 = 4.0 / (n_d2 * D)

    c = w_d2 * _huber_sum(rs - rt)
    for t_ref, s_ref in ((t_a_ref, s_a_ref), (t_b_ref, s_b_ref),
                         (t_c_ref, s_c_ref)):
        sh_t = t_ref[0, 0]
        sh_s = s_ref[0, 0]
        c += w_d1 * _huber_sum((rs - sh_s) - (rt - sh_t))
        c += w_d3 * _huber_sum(sh_s - sh_t)

    acc_ref[0] += c

    @pl.when(i == _N - 1)
    def _():
        out_ref[0, 0] = acc_ref[0]


def kernel(teacher_feats, student_feats):
    B, V, P, D = teacher_feats.shape
    tf = jax.lax.stop_gradient(teacher_feats)

    pk = jax.random.key(42)
    pk1, pk2 = jax.random.split(pk)
    ref_perm = jax.random.permutation(pk1, P)[:_N].astype(jnp.int32)
    shared_perm = jax.random.permutation(pk2, P)[:_N].astype(jnp.int32)

    tt = jnp.transpose(tf[:, ::2], (1, 2, 0, 3))
    st = jnp.transpose(student_feats, (1, 2, 0, 3))

    def spec(view, use_ref_perm):
        if use_ref_perm:
            return pl.BlockSpec((1, 1, B, D),
                                lambda i, rp, sp, v=view: (v, rp[i], 0, 0))
        return pl.BlockSpec((1, 1, B, D),
                            lambda i, rp, sp, v=view: (v, sp[i], 0, 0))

    in_specs = [spec(0, True), spec(0, True)]
    for v in (1, 2, 3):
        in_specs.append(spec(v, False))
        in_specs.append(spec(v, False))

    grid_spec = pltpu.PrefetchScalarGridSpec(
        num_scalar_prefetch=2,
        grid=(_N,),
        in_specs=in_specs,
        out_specs=pl.BlockSpec((1, 1), lambda i, rp, sp: (0, 0),
                               memory_space=pltpu.SMEM),
        scratch_shapes=[pltpu.SMEM((1,), jnp.float32)],
    )

    out = pl.pallas_call(
        _loss_kernel,
        grid_spec=grid_spec,
        out_shape=jax.ShapeDtypeStruct((1, 1), jnp.float32),
    )(ref_perm, shared_perm, tt, st, tt, st, tt, st, tt, st)
    return out[0, 0]
